# baseline (speedup 1.0000x reference)
import functools

import jax
import jax.numpy as jnp
from jax import lax
from jax.experimental import pallas as pl
from jax.experimental.pallas import tpu as pltpu
from jax.experimental.pallas import tpu_sc as plsc

BATCH = 16384
DIM = 128
L = 16
NC = 2
NS = 16
NW = NC * NS
BPW = BATCH // NW
GCH = 128
NG = BPW // GCH

ROWS_BLK = 512


def _gather_body(feat_hbm, w_hbm, out_hbm, idx_v, wg_v, sem_g):
    c = lax.axis_index("c")
    s = lax.axis_index("s")
    wid = s * NC + c
    pltpu.sync_copy(feat_hbm.at[wid], idx_v)
    gathers = [
        pltpu.async_copy(w_hbm.at[idx_v.at[j]], wg_v.at[pl.ds(j * GCH, GCH)], sem_g)
        for j in range(NG)
    ]
    for g in gathers:
        g.wait()
    pltpu.sync_copy(wg_v, out_hbm.at[pl.ds(wid * BPW, BPW)])


def _sc_gather(feat3d, weights_flat):
    mesh = plsc.VectorSubcoreMesh(core_axis_name="c", subcore_axis_name="s")
    kern = functools.partial(
        pl.kernel,
        mesh=mesh,
        out_type=jax.ShapeDtypeStruct((BATCH,), jnp.float32),
        scratch_types=[
            pltpu.VMEM((NG, GCH), jnp.int32),
            pltpu.VMEM((BPW,), jnp.float32),
            pltpu.SemaphoreType.DMA,
        ],
    )(_gather_body)
    return kern(feat3d, weights_flat)


NQ = 8
QROWS = BATCH // NQ
QSTEPS = QROWS // ROWS_BLK


def _norms_body(*refs):
    f_refs, o_refs = refs[:NQ], refs[NQ:]
    for f_ref, o_ref in zip(f_refs, o_refs):
        sq = f_ref[...] * f_ref[...]
        o_ref[...] = jnp.sum(jnp.transpose(sq), axis=0).reshape(1, ROWS_BLK)


def _tc_norms(factor):
    in_specs = [
        pl.BlockSpec((ROWS_BLK, DIM), (lambda i, _q=q: (_q * QSTEPS + i, 0)))
        for q in range(NQ)
    ]
    out_specs = [pl.BlockSpec((1, ROWS_BLK), lambda i: (0, i)) for _ in range(NQ)]
    return pl.pallas_call(
        _norms_body,
        grid=(QSTEPS,),
        in_specs=in_specs,
        out_specs=out_specs,
        out_shape=[jax.ShapeDtypeStruct((1, QROWS), jnp.float32)] * NQ,
    )(*([factor] * NQ))


def _dot_body(*refs):
    w_ref, n_refs, o_ref = refs[0], refs[1:1 + NQ], refs[1 + NQ]
    w = w_ref[...]
    total = jnp.float32(0.0)
    for q, n_ref in enumerate(n_refs):
        total = total + jnp.sum(w[:, q * QROWS:(q + 1) * QROWS] * n_ref[...])
    o_ref[0, 0] = total


def _tc_dot(w2d, norm_parts):
    return pl.pallas_call(
        _dot_body,
        in_specs=[pl.BlockSpec((1, BATCH), lambda: (0, 0))]
        + [pl.BlockSpec((1, QROWS), lambda: (0, 0)) for _ in range(NQ)],
        out_specs=pl.BlockSpec(memory_space=pltpu.SMEM),
        out_shape=jax.ShapeDtypeStruct((1, 1), jnp.float32),
    )(w2d, *norm_parts)


def kernel(factor, features, weights):
    feat3d = features.astype(jnp.int32).reshape(NW, NG, GCH)
    weights_flat = weights.reshape(-1)
    w = _sc_gather(feat3d, weights_flat)
    norm_parts = _tc_norms(factor)
    out = _tc_dot(w.reshape(1, BATCH), norm_parts)
    return out[0, 0]


# speedup vs baseline: 1.0341x; 1.0341x over previous
import functools

import jax
import jax.numpy as jnp
from jax import lax
from jax.experimental import pallas as pl
from jax.experimental.pallas import tpu as pltpu
from jax.experimental.pallas import tpu_sc as plsc

BATCH = 16384
DIM = 128
L = 16
NC = 2
NS = 16
NW = NC * NS
BPW = BATCH // NW
GCH = 128
NG = BPW // GCH

ROWS_BLK = 512


def _gather_body(feat_hbm, w_hbm, out_hbm, idx_v, wg_v, sem_g):
    c = lax.axis_index("c")
    s = lax.axis_index("s")
    wid = s * NC + c
    pltpu.sync_copy(feat_hbm.at[wid], idx_v)
    gathers = [
        pltpu.async_copy(w_hbm.at[idx_v.at[j]], wg_v.at[pl.ds(j * GCH, GCH)], sem_g)
        for j in range(NG)
    ]
    for g in gathers:
        g.wait()
    pltpu.sync_copy(wg_v, out_hbm.at[pl.ds(wid * BPW, BPW)])


def _sc_gather(feat3d, weights_flat):
    mesh = plsc.VectorSubcoreMesh(core_axis_name="c", subcore_axis_name="s")
    kern = functools.partial(
        pl.kernel,
        mesh=mesh,
        out_type=jax.ShapeDtypeStruct((BATCH,), jnp.float32),
        scratch_types=[
            pltpu.VMEM((NG, GCH), jnp.int32),
            pltpu.VMEM((BPW,), jnp.float32),
            pltpu.SemaphoreType.DMA,
        ],
    )(_gather_body)
    return kern(feat3d, weights_flat)


NQ = 16
QROWS = BATCH // NQ
QSTEPS = QROWS // ROWS_BLK


def _norms_body(*refs):
    f_refs, o_refs = refs[:NQ], refs[NQ:]
    for f_ref, o_ref in zip(f_refs, o_refs):
        sq = f_ref[...] * f_ref[...]
        o_ref[...] = jnp.sum(jnp.transpose(sq), axis=0).reshape(1, ROWS_BLK)


def _tc_norms(factor):
    in_specs = [
        pl.BlockSpec((ROWS_BLK, DIM), (lambda i, _q=q: (_q * QSTEPS + i, 0)))
        for q in range(NQ)
    ]
    out_specs = [pl.BlockSpec((1, ROWS_BLK), lambda i: (0, i)) for _ in range(NQ)]
    return pl.pallas_call(
        _norms_body,
        grid=(QSTEPS,),
        in_specs=in_specs,
        out_specs=out_specs,
        out_shape=[jax.ShapeDtypeStruct((1, QROWS), jnp.float32)] * NQ,
    )(*([factor] * NQ))


def _dot_body(*refs):
    w_ref, n_refs, o_ref = refs[0], refs[1:1 + NQ], refs[1 + NQ]
    w = w_ref[...]
    total = jnp.float32(0.0)
    for q, n_ref in enumerate(n_refs):
        total = total + jnp.sum(w[:, q * QROWS:(q + 1) * QROWS] * n_ref[...])
    o_ref[0, 0] = total


def _tc_dot(w2d, norm_parts):
    return pl.pallas_call(
        _dot_body,
        in_specs=[pl.BlockSpec((1, BATCH), lambda: (0, 0))]
        + [pl.BlockSpec((1, QROWS), lambda: (0, 0)) for _ in range(NQ)],
        out_specs=pl.BlockSpec(memory_space=pltpu.SMEM),
        out_shape=jax.ShapeDtypeStruct((1, 1), jnp.float32),
    )(w2d, *norm_parts)


def kernel(factor, features, weights):
    feat3d = features.astype(jnp.int32).reshape(NW, NG, GCH)
    weights_flat = weights.reshape(-1)
    w = _sc_gather(feat3d, weights_flat)
    norm_parts = _tc_norms(factor)
    out = _tc_dot(w.reshape(1, BATCH), norm_parts)
    return out[0, 0]


# speedup vs baseline: 1.0505x; 1.0159x over previous
import functools

import jax
import jax.numpy as jnp
from jax import lax
from jax.experimental import pallas as pl
from jax.experimental.pallas import tpu as pltpu
from jax.experimental.pallas import tpu_sc as plsc

BATCH = 16384
DIM = 128
L = 16
NC = 2
NS = 16
NW = NC * NS
BPW = BATCH // NW
GCH = 128
NG = BPW // GCH

ROWS_BLK = 1024


def _gather_body(feat_hbm, w_hbm, out_hbm, idx_v, wg_v, sem_g):
    c = lax.axis_index("c")
    s = lax.axis_index("s")
    wid = s * NC + c
    pltpu.sync_copy(feat_hbm.at[wid], idx_v)
    gathers = [
        pltpu.async_copy(w_hbm.at[idx_v.at[j]], wg_v.at[pl.ds(j * GCH, GCH)], sem_g)
        for j in range(NG)
    ]
    for g in gathers:
        g.wait()
    pltpu.sync_copy(wg_v, out_hbm.at[pl.ds(wid * BPW, BPW)])


def _sc_gather(feat3d, weights_flat):
    mesh = plsc.VectorSubcoreMesh(core_axis_name="c", subcore_axis_name="s")
    kern = functools.partial(
        pl.kernel,
        mesh=mesh,
        out_type=jax.ShapeDtypeStruct((BATCH,), jnp.float32),
        scratch_types=[
            pltpu.VMEM((NG, GCH), jnp.int32),
            pltpu.VMEM((BPW,), jnp.float32),
            pltpu.SemaphoreType.DMA,
        ],
    )(_gather_body)
    return kern(feat3d, weights_flat)


NQ = 8
QROWS = BATCH // NQ
QSTEPS = QROWS // ROWS_BLK


def _norms_body(*refs):
    f_refs, o_refs = refs[:NQ], refs[NQ:]
    for f_ref, o_ref in zip(f_refs, o_refs):
        sq = f_ref[...] * f_ref[...]
        o_ref[...] = jnp.sum(jnp.transpose(sq), axis=0).reshape(1, ROWS_BLK)


def _tc_norms(factor):
    in_specs = [
        pl.BlockSpec((ROWS_BLK, DIM), (lambda i, _q=q: (_q * QSTEPS + i, 0)))
        for q in range(NQ)
    ]
    out_specs = [pl.BlockSpec((1, ROWS_BLK), lambda i: (0, i)) for _ in range(NQ)]
    return pl.pallas_call(
        _norms_body,
        grid=(QSTEPS,),
        in_specs=in_specs,
        out_specs=out_specs,
        out_shape=[jax.ShapeDtypeStruct((1, QROWS), jnp.float32)] * NQ,
    )(*([factor] * NQ))


def _dot_body(*refs):
    w_ref, n_refs, o_ref = refs[0], refs[1:1 + NQ], refs[1 + NQ]
    w = w_ref[...]
    total = jnp.float32(0.0)
    for q, n_ref in enumerate(n_refs):
        total = total + jnp.sum(w[:, q * QROWS:(q + 1) * QROWS] * n_ref[...])
    o_ref[0, 0] = total


def _tc_dot(w2d, norm_parts):
    return pl.pallas_call(
        _dot_body,
        in_specs=[pl.BlockSpec((1, BATCH), lambda: (0, 0))]
        + [pl.BlockSpec((1, QROWS), lambda: (0, 0)) for _ in range(NQ)],
        out_specs=pl.BlockSpec(memory_space=pltpu.SMEM),
        out_shape=jax.ShapeDtypeStruct((1, 1), jnp.float32),
    )(w2d, *norm_parts)


def kernel(factor, features, weights):
    feat3d = features.astype(jnp.int32).reshape(NW, NG, GCH)
    weights_flat = weights.reshape(-1)
    w = _sc_gather(feat3d, weights_flat)
    norm_parts = _tc_norms(factor)
    out = _tc_dot(w.reshape(1, BATCH), norm_parts)
    return out[0, 0]


# speedup vs baseline: 1.0511x; 1.0005x over previous
import functools

import jax
import jax.numpy as jnp
from jax import lax
from jax.experimental import pallas as pl
from jax.experimental.pallas import tpu as pltpu
from jax.experimental.pallas import tpu_sc as plsc

BATCH = 16384
DIM = 128
L = 16
NC = 2
NS = 16
NW = NC * NS
BPW = BATCH // NW
GCH = 128
NG = BPW // GCH

ROWS_BLK = 1024


def _gather_body(feat_hbm, w_hbm, out_hbm, idx_v, wg_v, sem_g):
    c = lax.axis_index("c")
    s = lax.axis_index("s")
    wid = s * NC + c
    pltpu.sync_copy(feat_hbm.at[wid], idx_v)
    gathers = [
        pltpu.async_copy(w_hbm.at[idx_v.at[j]], wg_v.at[pl.ds(j * GCH, GCH)], sem_g)
        for j in range(NG)
    ]
    for g in gathers:
        g.wait()
    pltpu.sync_copy(wg_v, out_hbm.at[pl.ds(wid * BPW, BPW)])


def _sc_gather(feat3d, weights_flat):
    mesh = plsc.VectorSubcoreMesh(core_axis_name="c", subcore_axis_name="s")
    kern = functools.partial(
        pl.kernel,
        mesh=mesh,
        out_type=jax.ShapeDtypeStruct((BATCH,), jnp.float32),
        scratch_types=[
            pltpu.VMEM((NG, GCH), jnp.int32),
            pltpu.VMEM((BPW,), jnp.float32),
            pltpu.SemaphoreType.DMA,
        ],
    )(_gather_body)
    return kern(feat3d, weights_flat)


NQ = 8
QROWS = BATCH // NQ
QSTEPS = QROWS // ROWS_BLK


def _norms_body(*refs):
    f_refs, o_refs = refs[:NQ], refs[NQ:]
    for f_ref, o_ref in zip(f_refs, o_refs):
        sq = f_ref[...] * f_ref[...]
        o_ref[...] = jnp.sum(jnp.transpose(sq), axis=0).reshape(1, ROWS_BLK)


def _tc_norms(factor):
    in_specs = [
        pl.BlockSpec((ROWS_BLK, DIM), (lambda i, _q=q: (_q * QSTEPS + i, 0)))
        for q in range(NQ)
    ]
    out_specs = [pl.BlockSpec((1, ROWS_BLK), lambda i: (0, i)) for _ in range(NQ)]
    return pl.pallas_call(
        _norms_body,
        grid=(QSTEPS,),
        in_specs=in_specs,
        out_specs=out_specs,
        out_shape=[jax.ShapeDtypeStruct((1, QROWS), jnp.float32)] * NQ,
    )(*([factor] * NQ))


def _dot_body(*refs):
    w_ref, n_refs, o_ref = refs[0], refs[1:1 + NQ], refs[1 + NQ]
    w = w_ref[...]
    total = jnp.float32(0.0)
    for q, n_ref in enumerate(n_refs):
        total = total + jnp.sum(w[:, q * QROWS:(q + 1) * QROWS] * n_ref[...])
    o_ref[0, 0] = total


def _tc_dot(w2d, norm_parts):
    return pl.pallas_call(
        _dot_body,
        in_specs=[pl.BlockSpec((1, BATCH), lambda: (0, 0))]
        + [pl.BlockSpec((1, QROWS), lambda: (0, 0)) for _ in range(NQ)],
        out_specs=pl.BlockSpec(memory_space=pltpu.SMEM),
        out_shape=jax.ShapeDtypeStruct((1, 1), jnp.float32),
    )(w2d, *norm_parts)


def kernel(factor, features, weights):
    feat3d = features.astype(jnp.int32).reshape(NW, NG, GCH)
    weights_flat = weights.reshape(-1)
    norm_parts = _tc_norms(factor)
    w = _sc_gather(feat3d, weights_flat)
    out = _tc_dot(w.reshape(1, BATCH), norm_parts)
    return out[0, 0]
